# baseline (speedup 1.0000x reference)
import functools

import jax
import jax.numpy as jnp
from jax import lax
from jax.experimental import pallas as pl
from jax.experimental.pallas import tpu as pltpu
from jax.experimental.pallas import tpu_sc as plsc

PROMPT_LEN = 16
D = 768
SUF = 110
CTX = 1 + PROMPT_LEN + SUF
ROWS = 512
POOL2 = 2048 * PROMPT_LEN
NUM_CORES = 2
NUM_SUBCORES = 16
NW = NUM_CORES * NUM_SUBCORES
RPW = ROWS // NW
REP = 8
NIDX = RPW * PROMPT_LEN
SROWS = 7

_mesh = plsc.VectorSubcoreMesh(core_axis_name="c", subcore_axis_name="s")


@functools.partial(
    pl.kernel,
    out_type=(
        jax.ShapeDtypeStruct((CTX, ROWS, D), jnp.float32),
        jax.ShapeDtypeStruct((ROWS, CTX), jnp.int32),
    ),
    mesh=_mesh,
    scratch_types=[
        pltpu.VMEM((RPW,), jnp.int32),
        pltpu.VMEM((NIDX,), jnp.int32),
        pltpu.VMEM((3, RPW, D), jnp.float32),
        pltpu.VMEM((1, 1, D), jnp.float32),
        pltpu.VMEM_SHARED((SUF, REP, D), jnp.float32),
        pltpu.VMEM_SHARED((1, REP, D), jnp.float32),
        pltpu.VMEM_SHARED((REP, CTX), jnp.int32),
        pltpu.SemaphoreType.DMA,
        pltpu.SemaphoreType.DMA,
        pltpu.SemaphoreType.DMA,
        pltpu.SemaphoreType.DMA,
    ],
)
def _assemble(idx_hbm, prompt2_hbm, pre_hbm, suf_hbm, tok_hbm,
              out_emb, out_tok,
              idx_v, idx2_v, gbuf, bounce_v, suf_rep_sh, pre_rep_sh,
              tok_rep_sh, gsem, lsem, osem, wsem):
    cid = lax.axis_index("c")
    sid = lax.axis_index("s")
    wid = sid * NUM_CORES + cid
    base = wid * RPW

    start = sid * SROWS
    nrows = jnp.minimum(SROWS, SUF - start)

    def _stage_row(q):
        trow = start + q
        pltpu.sync_copy(suf_hbm.at[:, pl.ds(trow, 1)],
                        bounce_v.at[pl.ds(0, 1)])

        def _rep(rep, c):
            pltpu.sync_copy(bounce_v.at[pl.ds(0, 1)],
                            suf_rep_sh.at[pl.ds(trow, 1), pl.ds(rep, 1)])
            return c

        lax.fori_loop(0, REP, _rep, 0)

    pltpu.sync_copy(idx_hbm.at[pl.ds(base, RPW)], idx_v)

    idx16 = idx_v[...] * PROMPT_LEN
    for k in range(PROMPT_LEN):
        idx2_v[pl.ds(k * RPW, RPW)] = idx16 + k

    NB = 3
    gd = [None] * NB
    core_wr = [None] * NB
    for b in range(NB):
        gd[b] = pltpu.async_copy(
            prompt2_hbm.at[idx2_v.at[pl.ds(b * RPW, RPW)]], gbuf.at[b], gsem)
    for k in range(PROMPT_LEN):
        b = k % NB
        gd[b].wait()
        core_wr[b] = pltpu.async_copy(
            gbuf.at[pl.ds(b, 1)],
            out_emb.at[pl.ds(1 + k, 1), pl.ds(base, RPW)], osem)
        if k < SROWS:
            @pl.when(k < nrows)
            def _():
                _stage_row(k)
        if k < REP:
            @pl.when(sid == NUM_SUBCORES - 1)
            def _():
                pltpu.sync_copy(pre_hbm, pre_rep_sh.at[:, pl.ds(k, 1)])
                pltpu.sync_copy(tok_hbm, tok_rep_sh.at[pl.ds(k, 1)])
        if k + NB < PROMPT_LEN:
            core_wr[b].wait()
            gd[b] = pltpu.async_copy(
                prompt2_hbm.at[idx2_v.at[pl.ds((k + NB) * RPW, RPW)]],
                gbuf.at[b], gsem)

    plsc.subcore_barrier()

    tok_wr = [
        pltpu.async_copy(tok_rep_sh, out_tok.at[pl.ds(base + h * REP, REP)],
                         lsem)
        for h in range(RPW // REP)
    ]

    for h in range(RPW // REP):
        pltpu.make_async_copy(
            pre_rep_sh, out_emb.at[pl.ds(0, 1), pl.ds(base + h * REP, REP)],
            wsem).start()

    def _bc(t, c):
        for h in range(RPW // REP):
            pltpu.make_async_copy(
                suf_rep_sh.at[pl.ds(t, 1)],
                out_emb.at[pl.ds(1 + PROMPT_LEN + t, 1),
                           pl.ds(base + h * REP, REP)],
                wsem).start()
        return c

    lax.fori_loop(0, SUF, _bc, 0)

    def _drain(t, c):
        pltpu.make_async_copy(
            suf_hbm.at[:, pl.ds(0, REP)], suf_rep_sh.at[pl.ds(0, 1)],
            wsem).wait()
        return c

    lax.fori_loop(0, (SUF + 1) * (RPW // REP), _drain, 0)
    for b in range(NB):
        core_wr[(PROMPT_LEN - NB + b) % NB].wait()
    for w in tok_wr:
        w.wait()


def kernel(indices, mini_batch, prompt, embedding_prefix, embedding_suffix,
           tokenized_prompts):
    del mini_batch
    emb_t, tok = _assemble(
        indices.reshape(-1),
        prompt.reshape(POOL2, D),
        embedding_prefix, embedding_suffix, tokenized_prompts)
    return jnp.transpose(emb_t, (1, 0, 2)), tok


# speedup vs baseline: 1.0066x; 1.0066x over previous
import functools

import jax
import jax.numpy as jnp
from jax import lax
from jax.experimental import pallas as pl
from jax.experimental.pallas import tpu as pltpu
from jax.experimental.pallas import tpu_sc as plsc

PROMPT_LEN = 16
D = 768
SUF = 110
CTX = 1 + PROMPT_LEN + SUF
ROWS = 512
POOL2 = 2048 * PROMPT_LEN
NUM_CORES = 2
NUM_SUBCORES = 16
NW = NUM_CORES * NUM_SUBCORES
RPW = ROWS // NW
REP = 8
NIDX = RPW * PROMPT_LEN
SROWS = 7

_mesh = plsc.VectorSubcoreMesh(core_axis_name="c", subcore_axis_name="s")


@functools.partial(
    pl.kernel,
    out_type=(
        jax.ShapeDtypeStruct((CTX, ROWS, D), jnp.float32),
        jax.ShapeDtypeStruct((ROWS, CTX), jnp.int32),
    ),
    mesh=_mesh,
    scratch_types=[
        pltpu.VMEM((RPW,), jnp.int32),
        pltpu.VMEM((NIDX,), jnp.int32),
        pltpu.VMEM((3, RPW, D), jnp.float32),
        pltpu.VMEM((1, 1, D), jnp.float32),
        pltpu.VMEM_SHARED((SUF, REP, D), jnp.float32),
        pltpu.VMEM_SHARED((1, REP, D), jnp.float32),
        pltpu.VMEM_SHARED((REP, CTX), jnp.int32),
        pltpu.SemaphoreType.DMA,
        pltpu.SemaphoreType.DMA,
        pltpu.SemaphoreType.DMA,
        pltpu.SemaphoreType.DMA,
    ],
)
def _assemble(idx_hbm, prompt2_hbm, pre_hbm, suf_hbm, tok_hbm,
              out_emb, out_tok,
              idx_v, idx2_v, gbuf, bounce_v, suf_rep_sh, pre_rep_sh,
              tok_rep_sh, gsem, lsem, osem, wsem):
    cid = lax.axis_index("c")
    sid = lax.axis_index("s")
    wid = sid * NUM_CORES + cid
    base = wid * RPW

    start = sid * SROWS
    nrows = jnp.minimum(SROWS, SUF - start)

    def _stage_row(q, carry):
        trow = start + q
        pltpu.sync_copy(suf_hbm.at[:, pl.ds(trow, 1)],
                        bounce_v.at[pl.ds(0, 1)])

        def _rep(rep, c):
            pltpu.sync_copy(bounce_v.at[pl.ds(0, 1)],
                            suf_rep_sh.at[pl.ds(trow, 1), pl.ds(rep, 1)])
            return c

        return lax.fori_loop(0, REP, _rep, carry)

    lax.fori_loop(0, nrows, _stage_row, 0)

    @pl.when(sid == NUM_SUBCORES - 1)
    def _():
        for rep in range(REP):
            pltpu.sync_copy(pre_hbm, pre_rep_sh.at[:, pl.ds(rep, 1)])
            pltpu.sync_copy(tok_hbm, tok_rep_sh.at[pl.ds(rep, 1)])

    pltpu.sync_copy(idx_hbm.at[pl.ds(base, RPW)], idx_v)

    idx16 = idx_v[...] * PROMPT_LEN
    for k in range(PROMPT_LEN):
        idx2_v[pl.ds(k * RPW, RPW)] = idx16 + k

    plsc.subcore_barrier()

    tok_wr = [
        pltpu.async_copy(tok_rep_sh, out_tok.at[pl.ds(base + h * REP, REP)],
                         lsem)
        for h in range(RPW // REP)
    ]

    for h in range(RPW // REP):
        pltpu.make_async_copy(
            pre_rep_sh, out_emb.at[pl.ds(0, 1), pl.ds(base + h * REP, REP)],
            wsem).start()

    def _bc(t, c):
        for h in range(RPW // REP):
            pltpu.make_async_copy(
                suf_rep_sh.at[pl.ds(t, 1)],
                out_emb.at[pl.ds(1 + PROMPT_LEN + t, 1),
                           pl.ds(base + h * REP, REP)],
                wsem).start()
        return c

    lax.fori_loop(0, SUF, _bc, 0)

    NB = 3
    gd = [None] * NB
    core_wr = [None] * NB
    for b in range(NB):
        gd[b] = pltpu.async_copy(
            prompt2_hbm.at[idx2_v.at[pl.ds(b * RPW, RPW)]], gbuf.at[b], gsem)
    for k in range(PROMPT_LEN):
        b = k % NB
        gd[b].wait()
        core_wr[b] = pltpu.async_copy(
            gbuf.at[pl.ds(b, 1)],
            out_emb.at[pl.ds(1 + k, 1), pl.ds(base, RPW)], osem)
        if k + NB < PROMPT_LEN:
            core_wr[b].wait()
            gd[b] = pltpu.async_copy(
                prompt2_hbm.at[idx2_v.at[pl.ds((k + NB) * RPW, RPW)]],
                gbuf.at[b], gsem)

    def _drain(t, c):
        pltpu.make_async_copy(
            suf_hbm.at[:, pl.ds(0, REP)], suf_rep_sh.at[pl.ds(0, 1)],
            wsem).wait()
        return c

    lax.fori_loop(0, (SUF + 1) * (RPW // REP), _drain, 0)
    for b in range(NB):
        core_wr[(PROMPT_LEN - NB + b) % NB].wait()
    for w in tok_wr:
        w.wait()


def kernel(indices, mini_batch, prompt, embedding_prefix, embedding_suffix,
           tokenized_prompts):
    del mini_batch
    emb_t, tok = _assemble(
        indices.reshape(-1),
        prompt.reshape(POOL2, D),
        embedding_prefix, embedding_suffix, tokenized_prompts)
    return jnp.transpose(emb_t, (1, 0, 2)), tok


# speedup vs baseline: 1.0322x; 1.0254x over previous
import functools

import jax
import jax.numpy as jnp
from jax import lax
from jax.experimental import pallas as pl
from jax.experimental.pallas import tpu as pltpu
from jax.experimental.pallas import tpu_sc as plsc

PROMPT_LEN = 16
D = 768
SUF = 110
CTX = 1 + PROMPT_LEN + SUF
ROWS = 512
POOL2 = 2048 * PROMPT_LEN
NUM_CORES = 2
NUM_SUBCORES = 16
NW = NUM_CORES * NUM_SUBCORES
RPW = ROWS // NW
REP = 8
NIDX = RPW * PROMPT_LEN
SROWS = 7

_mesh = plsc.VectorSubcoreMesh(core_axis_name="c", subcore_axis_name="s")


@functools.partial(
    pl.kernel,
    out_type=(
        jax.ShapeDtypeStruct((CTX, ROWS, D), jnp.float32),
        jax.ShapeDtypeStruct((ROWS, CTX), jnp.int32),
    ),
    mesh=_mesh,
    scratch_types=[
        pltpu.VMEM((RPW,), jnp.int32),
        pltpu.VMEM((NIDX,), jnp.int32),
        pltpu.VMEM((3, RPW, D), jnp.float32),
        pltpu.VMEM((1, REP, D), jnp.float32),
        pltpu.VMEM((2, REP, D), jnp.float32),
        pltpu.VMEM_SHARED((SUF, REP, D), jnp.float32),
        pltpu.VMEM_SHARED((1, REP, D), jnp.float32),
        pltpu.VMEM_SHARED((REP, CTX), jnp.int32),
        pltpu.SemaphoreType.DMA,
        pltpu.SemaphoreType.DMA,
        pltpu.SemaphoreType.DMA,
        pltpu.SemaphoreType.DMA,
    ],
)
def _assemble(idx_hbm, prompt2_hbm, pre_hbm, suf_hbm, tok_hbm,
              out_emb, out_tok,
              idx_v, idx2_v, gbuf, bounce_v, rep_v, suf_rep_sh, pre_rep_sh,
              tok_rep_sh, gsem, lsem, osem, wsem):
    cid = lax.axis_index("c")
    sid = lax.axis_index("s")
    wid = sid * NUM_CORES + cid
    base = wid * RPW

    def _replicate(src_row, dst_sh, trow, buf):
        def _cp(i, c):
            v = bounce_v[0, src_row, pl.ds(i * 16, 16)]
            for rep in range(REP):
                rep_v[buf, rep, pl.ds(i * 16, 16)] = v
            return c

        lax.fori_loop(0, D // 16, _cp, 0)
        return pltpu.async_copy(rep_v.at[pl.ds(buf, 1)],
                                dst_sh.at[pl.ds(trow, 1)], lsem)

    BLK = [min(REP, SUF - REP * s) for s in range(14)]

    for s in range(13):
        @pl.when(sid == s)
        def _():
            pltpu.sync_copy(suf_hbm.at[:, pl.ds(REP * s, REP)], bounce_v)
            wr = None
            for q in range(REP):
                if wr is not None:
                    wr.wait()
                wr = _replicate(q, suf_rep_sh, REP * s + q, q % 2)
            wr.wait()

    @pl.when(sid == 13)
    def _():
        for q in range(6):
            pltpu.sync_copy(suf_hbm.at[:, pl.ds(104 + q, 1)],
                            bounce_v.at[:, pl.ds(q, 1)])
        wr = None
        for q in range(6):
            if wr is not None:
                wr.wait()
            wr = _replicate(q, suf_rep_sh, 104 + q, q % 2)
        wr.wait()

    @pl.when(sid == 14)
    def _():
        pltpu.sync_copy(pre_hbm, bounce_v.at[:, pl.ds(0, 1)])
        _replicate(0, pre_rep_sh, 0, 0).wait()

    @pl.when(sid == 15)
    def _():
        for rep in range(REP):
            pltpu.sync_copy(tok_hbm, tok_rep_sh.at[pl.ds(rep, 1)])

    pltpu.sync_copy(idx_hbm.at[pl.ds(base, RPW)], idx_v)

    idx16 = idx_v[...] * PROMPT_LEN
    for k in range(PROMPT_LEN):
        idx2_v[pl.ds(k * RPW, RPW)] = idx16 + k

    plsc.subcore_barrier()

    tok_wr = [
        pltpu.async_copy(tok_rep_sh, out_tok.at[pl.ds(base + h * REP, REP)],
                         lsem)
        for h in range(RPW // REP)
    ]

    for h in range(RPW // REP):
        pltpu.make_async_copy(
            pre_rep_sh, out_emb.at[pl.ds(0, 1), pl.ds(base + h * REP, REP)],
            wsem).start()

    def _bc(t, c):
        for h in range(RPW // REP):
            pltpu.make_async_copy(
                suf_rep_sh.at[pl.ds(t, 1)],
                out_emb.at[pl.ds(1 + PROMPT_LEN + t, 1),
                           pl.ds(base + h * REP, REP)],
                wsem).start()
        return c

    lax.fori_loop(0, SUF, _bc, 0)

    NB = 3
    gd = [None] * NB
    core_wr = [None] * NB
    for b in range(NB):
        gd[b] = pltpu.async_copy(
            prompt2_hbm.at[idx2_v.at[pl.ds(b * RPW, RPW)]], gbuf.at[b], gsem)
    for k in range(PROMPT_LEN):
        b = k % NB
        gd[b].wait()
        core_wr[b] = pltpu.async_copy(
            gbuf.at[pl.ds(b, 1)],
            out_emb.at[pl.ds(1 + k, 1), pl.ds(base, RPW)], osem)
        if k + NB < PROMPT_LEN:
            core_wr[b].wait()
            gd[b] = pltpu.async_copy(
                prompt2_hbm.at[idx2_v.at[pl.ds((k + NB) * RPW, RPW)]],
                gbuf.at[b], gsem)

    def _drain(t, c):
        pltpu.make_async_copy(
            suf_hbm.at[:, pl.ds(0, REP)], suf_rep_sh.at[pl.ds(0, 1)],
            wsem).wait()
        return c

    lax.fori_loop(0, (SUF + 1) * (RPW // REP), _drain, 0)
    for b in range(NB):
        core_wr[(PROMPT_LEN - NB + b) % NB].wait()
    for w in tok_wr:
        w.wait()


def kernel(indices, mini_batch, prompt, embedding_prefix, embedding_suffix,
           tokenized_prompts):
    del mini_batch
    emb_t, tok = _assemble(
        indices.reshape(-1),
        prompt.reshape(POOL2, D),
        embedding_prefix, embedding_suffix, tokenized_prompts)
    return jnp.transpose(emb_t, (1, 0, 2)), tok
